# Initial kernel scaffold; baseline (speedup 1.0000x reference)
import jax
import jax.numpy as jnp
from jax.experimental import pallas as pl


def kernel(x, emb, g):
    raise NotImplementedError("write your pallas kernel here")


# speedup vs baseline: 5.2755x; 5.2755x over previous
import functools

import jax
import jax.numpy as jnp
from jax import lax
from jax.experimental import pallas as pl
from jax.experimental.pallas import tpu as pltpu, tpu_sc as plsc

_V = 8192
_D = 64
_B = 4096
_S = 200
_N = _B * _S
_NW = 32
_PER_W = _N // _NW
_CHUNK = 1280
_NCH = _PER_W // _CHUNK
_JROWS = 128
_NJ = _CHUNK // _JROWS
_ROWS_PER_TILE = _V // 16
_MULT = 1315423911


def _body(xp_hbm, tab_hbm, g_hbm, out_hbm, cur_v, idx_v, rows_v, g_v, tab_sh, sem):
    c = lax.axis_index("c")
    s = lax.axis_index("s")
    wid = s * 2 + c
    lane = lax.iota(jnp.int32, 16)

    pltpu.sync_copy(g_hbm, g_v)
    sg = []
    for c4 in range(4):
        gv = g_v[pl.ds(c4 * 16, 16)]
        sg.append(1.0 / (1.0 + jnp.exp(-gv)))

    row0 = s * jnp.int32(_ROWS_PER_TILE)
    pltpu.sync_copy(tab_hbm.at[pl.ds(row0, _ROWS_PER_TILE)],
                    rows_v.at[pl.ds(0, _ROWS_PER_TILE)])

    def _scale_row(r, _):
        for c4 in range(4):
            rows_v[r, pl.ds(c4 * 16, 16)] = rows_v[r, pl.ds(c4 * 16, 16)] * sg[c4]
        return 0

    lax.fori_loop(jnp.int32(0), jnp.int32(_ROWS_PER_TILE), _scale_row, 0)
    pltpu.sync_copy(rows_v.at[pl.ds(0, _ROWS_PER_TILE)],
                    tab_sh.at[pl.ds(row0, _ROWS_PER_TILE)])
    plsc.subcore_barrier()

    def _chunk(t, _):
        base = wid * jnp.int32(_PER_W) + t * jnp.int32(_CHUNK)
        pltpu.sync_copy(xp_hbm.at[pl.ds(base, _CHUNK + 16)], cur_v)
        for j in range(_NJ):
            def _hash(kk, _, j=j):
                pos0 = jnp.int32(j * _JROWS) + kk * jnp.int32(16)
                cur = cur_v[pl.ds(pos0 + 8, 16)]
                prevraw = cur_v[pl.ds(pos0 + 7, 16)]
                p = (base + pos0) + lane
                prev = jnp.where(p % _S == 0, 0, prevraw)
                idx_v[jnp.int32(j), pl.ds(kk * jnp.int32(16), 16)] = (
                    prev * _MULT + cur) & (_V - 1)
                return 0

            lax.fori_loop(jnp.int32(0), jnp.int32(_JROWS // 16), _hash, 0)
        copies = [
            pltpu.async_copy(tab_sh.at[idx_v.at[jnp.int32(j)]],
                             rows_v.at[pl.ds(j * _JROWS, _JROWS)], sem)
            for j in range(_NJ)
        ]
        for cp in copies:
            cp.wait()
        pltpu.sync_copy(rows_v, out_hbm.at[pl.ds(base, _CHUNK)])
        return 0

    lax.fori_loop(jnp.int32(0), jnp.int32(_NCH), _chunk, 0)


_call = pl.kernel(
    _body,
    out_type=jax.ShapeDtypeStruct((_N, _D), jnp.float32),
    mesh=plsc.VectorSubcoreMesh(core_axis_name="c", subcore_axis_name="s"),
    scratch_types=[
        pltpu.VMEM((_CHUNK + 16,), jnp.int32),
        pltpu.VMEM((_NJ, _JROWS), jnp.int32),
        pltpu.VMEM((_CHUNK, _D), jnp.float32),
        pltpu.VMEM((_D,), jnp.float32),
        pltpu.VMEM_SHARED((_V, _D), jnp.float32),
        pltpu.SemaphoreType.DMA,
    ],
    compiler_params=pltpu.CompilerParams(use_tc_tiling_on_sc=False),
)


@jax.jit
def kernel(x, emb, g):
    x32 = x.reshape(-1).astype(jnp.int32)
    zpad = jnp.zeros((8,), jnp.int32)
    xp = jnp.concatenate([zpad, x32, zpad])
    out = _call(xp, emb.astype(jnp.float32), g.astype(jnp.float32))
    return out.reshape(_B, _S, _D)


# speedup vs baseline: 5.9789x; 1.1333x over previous
import functools

import jax
import jax.numpy as jnp
from jax import lax
from jax.experimental import pallas as pl
from jax.experimental.pallas import tpu as pltpu, tpu_sc as plsc

_V = 8192
_D = 64
_B = 4096
_S = 200
_N = _B * _S
_NW = 32
_PER_W = _N // _NW
_CHUNK = 400
_NCH = _PER_W // _CHUNK
_JROWS = 80
_NJ = _CHUNK // _JROWS
_ROWS_PER_TILE = _V // 16
_MULT = 1315423911


def _make_body(stride):
    def _body(xs_hbm, tab_hbm, g_hbm, out_hbm,
              cur_v, idx_v, rows_v, g_v, tab_sh, sem_g, sem_x, sem_s):
        c = lax.axis_index("c")
        s = lax.axis_index("s")
        wid = s * 2 + c
        lane = lax.iota(jnp.int32, 16)
        i32 = jnp.int32

        pltpu.sync_copy(g_hbm, g_v)
        sg = []
        for c4 in range(4):
            gv = g_v[pl.ds(c4 * 16, 16)]
            sg.append(1.0 / (1.0 + jnp.exp(-gv)))

        row0 = s * i32(_ROWS_PER_TILE)
        pltpu.sync_copy(tab_hbm.at[pl.ds(row0, _ROWS_PER_TILE)],
                        rows_v.at[i32(0), pl.ds(0, _ROWS_PER_TILE)])

        def _scale_row(r, _):
            for c4 in range(4):
                rows_v[i32(0), r, pl.ds(c4 * 16, 16)] = (
                    rows_v[i32(0), r, pl.ds(c4 * 16, 16)] * sg[c4])
            return 0

        lax.fori_loop(i32(0), i32(_ROWS_PER_TILE), _scale_row, 0)
        pltpu.sync_copy(rows_v.at[i32(0), pl.ds(0, _ROWS_PER_TILE)],
                        tab_sh.at[pl.ds(row0, _ROWS_PER_TILE)])
        plsc.subcore_barrier()

        def _chunk(t, _):
            b = t & i32(1)
            nb = i32(1) - b
            base = wid * i32(_PER_W) + t * i32(_CHUNK)

            cs = _CHUNK * stride
            boff = b * i32(cs)

            @pl.when(t == i32(0))
            def _prime():
                pltpu.async_copy(
                    xs_hbm.at[pl.ds(base * i32(stride), cs)],
                    cur_v.at[pl.ds(boff, cs)], sem_x.at[b])

            pltpu.make_async_copy(
                xs_hbm.at[pl.ds(0, cs)], cur_v.at[pl.ds(boff, cs)],
                sem_x.at[b]).wait()

            @pl.when(t + i32(1) < i32(_NCH))
            def _prefetch():
                pltpu.async_copy(
                    xs_hbm.at[pl.ds((base + i32(_CHUNK)) * i32(stride), cs)],
                    cur_v.at[pl.ds(i32(cs) - boff, cs)], sem_x.at[nb])

            def _hash(kk, _):
                pos0 = kk * i32(16)
                pos = pos0 + lane
                if stride == 1:
                    cur = cur_v[pl.ds(boff + pos0, 16)]
                    prevraw = plsc.load_gather(
                        cur_v, [boff + jnp.maximum(pos - 1, 0)])
                else:
                    cur = (plsc.load_gather(cur_v, [boff + 2 * pos])
                           | plsc.load_gather(cur_v, [boff + 2 * pos + 1]))
                    prevraw = (
                        plsc.load_gather(
                            cur_v, [boff + jnp.maximum(2 * pos - 2, 0)])
                        | plsc.load_gather(
                            cur_v, [boff + jnp.maximum(2 * pos - 1, 0)]))
                col0 = ((base + pos) % i32(_S)) == i32(0)
                prev = jnp.where(col0, 0, prevraw)
                idx_v[b, pl.ds(pos0, 16)] = (prev * i32(_MULT) + cur) & i32(_V - 1)
                return 0

            lax.fori_loop(i32(0), i32(_CHUNK // 16), _hash, 0)

            @pl.when(t >= i32(2))
            def _drain_scatter():
                pltpu.make_async_copy(
                    rows_v.at[b], out_hbm.at[pl.ds(0, _CHUNK)],
                    sem_s.at[b]).wait()

            copies = [
                pltpu.async_copy(
                    tab_sh.at[idx_v.at[b, pl.ds(j * _JROWS, _JROWS)]],
                    rows_v.at[b, pl.ds(j * _JROWS, _JROWS)], sem_g)
                for j in range(_NJ)
            ]
            for cp in copies:
                cp.wait()

            pltpu.async_copy(rows_v.at[b], out_hbm.at[pl.ds(base, _CHUNK)],
                             sem_s.at[b])
            return 0

        lax.fori_loop(i32(0), i32(_NCH), _chunk, 0)

        for bb in range(2):
            pltpu.make_async_copy(
                rows_v.at[jnp.int32(bb)], out_hbm.at[pl.ds(0, _CHUNK)],
                sem_s.at[jnp.int32(bb)]).wait()

    return _body


def _make_call(stride):
    return pl.kernel(
        _make_body(stride),
        out_type=jax.ShapeDtypeStruct((_N, _D), jnp.float32),
        mesh=plsc.VectorSubcoreMesh(core_axis_name="c", subcore_axis_name="s"),
        scratch_types=[
            pltpu.VMEM((2 * _CHUNK * stride,), jnp.int32),
            pltpu.VMEM((2, _CHUNK), jnp.int32),
            pltpu.VMEM((2, _CHUNK, _D), jnp.float32),
            pltpu.VMEM((_D,), jnp.float32),
            pltpu.VMEM_SHARED((_V, _D), jnp.float32),
            pltpu.SemaphoreType.DMA,
            pltpu.SemaphoreType.DMA((2,)),
            pltpu.SemaphoreType.DMA((2,)),
        ],
        compiler_params=pltpu.CompilerParams(use_tc_tiling_on_sc=False,
                                             needs_layout_passes=False),
    )


_call_s1 = _make_call(1)


@jax.jit
def kernel(x, emb, g):
    xs = x.astype(jnp.int32).reshape(-1)
    out = _call_s1(xs, emb.astype(jnp.float32), g.astype(jnp.float32))
    return out.reshape(_B, _S, _D)
